# Initial kernel scaffold; baseline (speedup 1.0000x reference)
import jax
import jax.numpy as jnp
from jax.experimental import pallas as pl


def kernel(x, edge_index, edge_attr, W1, b1, W2, b2):
    raise NotImplementedError("write your pallas kernel here")


# speedup vs baseline: 34.6708x; 34.6708x over previous
import functools

import jax
import jax.numpy as jnp
from jax import lax
from jax.experimental import pallas as pl
from jax.experimental.pallas import tpu as pltpu
from jax.experimental.pallas import tpu_sc as plsc

N = 10000
E = 320000
D = 128
DE = 16
H = 64
R = 5

NC, NS = 2, 16
NW = NC * NS
EW = E // NW
GC = 128
OC = 1280
NOC = 8
EWP = NOC * OC
EP = (NW - 1) * EW + EWP

EC = E // NS
HALF = EC // NC
NR = (N + 127) // 128
NPAD = NR * 128

EB = 2000


def _ab_body(x_ref, wa_ref, wb_ref, a_ref, b_ref):
    xb = x_ref[...]
    a_ref[...] = lax.dot(xb, wa_ref[...], preferred_element_type=jnp.float32)
    b_ref[...] = lax.dot(xb, wb_ref[...], preferred_element_type=jnp.float32)


def _score_body(g_ref, ea_ref, wc_ref, b1_ref, w2_ref, b2_ref, p_ref):
    h = g_ref[...] + lax.dot(ea_ref[...], wc_ref[...],
                             preferred_element_type=jnp.float32) + b1_ref[...]
    h = jnp.maximum(h, 0.0)
    s = lax.dot(h, w2_ref[...], preferred_element_type=jnp.float32) + b2_ref[...]
    p_ref[...] = jax.nn.sigmoid(s)


_sc_mesh = plsc.VectorSubcoreMesh(core_axis_name="c", subcore_axis_name="s")
_sc_params = pltpu.CompilerParams(use_tc_tiling_on_sc=False,
                                  needs_layout_passes=False)


@functools.partial(
    pl.kernel, mesh=_sc_mesh, compiler_params=_sc_params,
    out_type=jax.ShapeDtypeStruct((EP, H), jnp.float32),
    scratch_types=[
        pltpu.VMEM((EWP,), jnp.int32),
        pltpu.VMEM((EWP,), jnp.int32),
        pltpu.VMEM((OC, H), jnp.float32),
        pltpu.SemaphoreType.DMA,
    ],
)
def _gather_ab(a_hbm, b_hbm, src_hbm, dst_hbm, g_hbm, idxs_v, idxd_v, rows_v, sem):
    wid = lax.axis_index("s") * NC + lax.axis_index("c")
    base = wid * EW
    pltpu.sync_copy(src_hbm.at[pl.ds(base, EWP)], idxs_v)
    pltpu.sync_copy(dst_hbm.at[pl.ds(base, EWP)], idxd_v)
    for oc in range(NOC):
        descs = [
            pltpu.async_copy(
                a_hbm.at[idxs_v.at[pl.ds(oc * OC + j * GC, GC)]],
                rows_v.at[pl.ds(j * GC, GC)], sem)
            for j in range(OC // GC)
        ]
        for d in descs:
            d.wait()
        descs = [
            pltpu.async_copy(
                b_hbm.at[idxd_v.at[pl.ds(oc * OC + j * GC, GC)]],
                rows_v.at[pl.ds(j * GC, GC)], sem, add=True)
            for j in range(OC // GC)
        ]
        for d in descs:
            d.wait()
        pltpu.sync_copy(rows_v, g_hbm.at[pl.ds(base + oc * OC, OC)])


@functools.partial(
    pl.kernel, mesh=_sc_mesh, compiler_params=_sc_params,
    out_type=jax.ShapeDtypeStruct((R, E), jnp.float32),
    scratch_types=[
        pltpu.VMEM((EC,), jnp.int32),
        pltpu.VMEM((EC,), jnp.int32),
        pltpu.VMEM((EC,), jnp.float32),
        pltpu.VMEM((EC,), jnp.float32),
        pltpu.VMEM((NR, 128), jnp.float32),
        pltpu.VMEM((NR, 128), jnp.float32),
        pltpu.VMEM((NR,), jnp.int32),
        pltpu.VMEM_SHARED((NR, 128), jnp.float32),
    ],
)
def _frontier(src_hbm, dst_hbm, p_hbm, m0_hbm, rows_hbm, out_hbm,
              srcv, dstv, pv, pmv, mloc, updv, rowids, sm):
    cid = lax.axis_index("c")
    sid = lax.axis_index("s")
    tbase = sid * EC
    pltpu.sync_copy(src_hbm.at[pl.ds(tbase, EC)], srcv)
    pltpu.sync_copy(dst_hbm.at[pl.ds(tbase, EC)], dstv)
    pltpu.sync_copy(p_hbm.at[pl.ds(tbase, EC)], pv)
    pltpu.sync_copy(m0_hbm, mloc)
    pltpu.sync_copy(rows_hbm, rowids)

    @pl.when(sid == 0)
    def _():
        pltpu.sync_copy(m0_hbm, sm)

    plsc.subcore_barrier()

    zeros = jnp.zeros((16,), jnp.float32)
    ones = jnp.ones((16,), jnp.float32)

    for r in range(R):
        def zbody(i, _):
            updv[i >> 3, pl.ds((i & 7) * 16, 16)] = zeros
            return 0
        lax.fori_loop(0, NR * 8, zbody, 0)

        def ebody(k, _):
            off = k * 16
            sv = srcv[pl.ds(off, 16)]
            dv = dstv[pl.ds(off, 16)]
            ppv = pv[pl.ds(off, 16)]
            ms = plsc.load_gather(mloc, [sv >> 7, sv & 127])
            md = plsc.load_gather(mloc, [dv >> 7, dv & 127])
            pm = ppv * ms * (1.0 - md)
            pmv[pl.ds(off, 16)] = pm
            plsc.store_scatter(updv, [dv >> 7, dv & 127], ones, mask=pm > 0.5)
            return 0
        lax.fori_loop(0, EC // 16, ebody, 0)

        pltpu.sync_copy(pmv.at[pl.ds(cid * HALF, HALF)],
                        out_hbm.at[r, pl.ds(tbase + cid * HALF, HALF)])

        if r < R - 1:
            pltpu.sync_copy(updv, sm.at[rowids], add=True)
            plsc.subcore_barrier()
            pltpu.sync_copy(sm, mloc)
            plsc.subcore_barrier()

            def cbody(i, _):
                v = mloc[i >> 3, pl.ds((i & 7) * 16, 16)]
                mloc[i >> 3, pl.ds((i & 7) * 16, 16)] = jnp.where(v > 0.5, 1.0, 0.0)
                return 0
            lax.fori_loop(0, NR * 8, cbody, 0)


def kernel(x, edge_index, edge_attr, W1, b1, W2, b2):
    src = edge_index[0]
    dst = edge_index[1]
    w1a = W1[:D]
    w1b = W1[D:2 * D]
    w1c = W1[2 * D:]

    a_tab, b_tab = pl.pallas_call(
        _ab_body,
        grid=(N // 1000,),
        in_specs=[
            pl.BlockSpec((1000, D), lambda i: (i, 0)),
            pl.BlockSpec((D, H), lambda i: (0, 0)),
            pl.BlockSpec((D, H), lambda i: (0, 0)),
        ],
        out_specs=[
            pl.BlockSpec((1000, H), lambda i: (i, 0)),
            pl.BlockSpec((1000, H), lambda i: (i, 0)),
        ],
        out_shape=[
            jax.ShapeDtypeStruct((N, H), jnp.float32),
            jax.ShapeDtypeStruct((N, H), jnp.float32),
        ],
    )(x, w1a, w1b)

    pad = jnp.zeros((EP - E,), jnp.int32)
    srcp = jnp.concatenate([src, pad])
    dstp = jnp.concatenate([dst, pad])
    g = _gather_ab(a_tab, b_tab, srcp, dstp)

    p2 = pl.pallas_call(
        _score_body,
        grid=(E // EB,),
        in_specs=[
            pl.BlockSpec((EB, H), lambda i: (i, 0)),
            pl.BlockSpec((EB, DE), lambda i: (i, 0)),
            pl.BlockSpec((DE, H), lambda i: (0, 0)),
            pl.BlockSpec((1, H), lambda i: (0, 0)),
            pl.BlockSpec((H, 1), lambda i: (0, 0)),
            pl.BlockSpec((1, 1), lambda i: (0, 0)),
        ],
        out_specs=pl.BlockSpec((EB, 1), lambda i: (i, 0)),
        out_shape=jax.ShapeDtypeStruct((E, 1), jnp.float32),
    )(g, edge_attr, w1c, b1.reshape(1, H), W2, b2.reshape(1, 1))
    p = p2.reshape(E)

    m0 = jnp.where(jnp.arange(NPAD) % 10 == 0, 1.0, 0.0)
    m0 = m0.astype(jnp.float32).reshape(NR, 128)
    rowids = jnp.arange(NR, dtype=jnp.int32)
    return _frontier(src, dst, p, m0, rowids)


# speedup vs baseline: 45.0105x; 1.2982x over previous
import functools

import jax
import jax.numpy as jnp
from jax import lax
from jax.experimental import pallas as pl
from jax.experimental.pallas import tpu as pltpu
from jax.experimental.pallas import tpu_sc as plsc

N = 10000
E = 320000
D = 128
DE = 16
H = 64
R = 5

NC, NS = 2, 16
NW = NC * NS
EW = E // NW
GC = 128
OC = 1280
NOC = 8
EWP = NOC * OC
EP = (NW - 1) * EW + EWP

EC = E // NS
HALF = EC // NC
NR = (N + 127) // 128
NPAD = NR * 128

EB = 4000


def _ab_body(x_ref, wa_ref, wb_ref, a_ref, b_ref):
    xb = x_ref[...]
    a_ref[...] = lax.dot(xb, wa_ref[...], preferred_element_type=jnp.float32)
    b_ref[...] = lax.dot(xb, wb_ref[...], preferred_element_type=jnp.float32)


def _score_body(g_ref, ea_ref, wc_ref, b1_ref, w2_ref, b2_ref, p_ref):
    h = g_ref[:, :H] + lax.dot(ea_ref[...], wc_ref[...],
                               preferred_element_type=jnp.float32) + b1_ref[...]
    h = jnp.maximum(h, 0.0)
    s = lax.dot(h, w2_ref[...], preferred_element_type=jnp.float32) + b2_ref[...]
    p_ref[...] = jax.nn.sigmoid(s)


_sc_mesh = plsc.VectorSubcoreMesh(core_axis_name="c", subcore_axis_name="s")
_sc_params = pltpu.CompilerParams(use_tc_tiling_on_sc=False,
                                  needs_layout_passes=False)


@functools.partial(
    pl.kernel, mesh=_sc_mesh, compiler_params=_sc_params,
    out_type=jax.ShapeDtypeStruct((EP, 2 * H), jnp.float32),
    scratch_types=[
        pltpu.VMEM((EWP,), jnp.int32),
        pltpu.VMEM((EWP,), jnp.int32),
        pltpu.VMEM((OC, H), jnp.float32),
        pltpu.SemaphoreType.DMA,
    ],
)
def _gather_ab(a_hbm, b_hbm, src_hbm, dst_hbm, g_hbm, idxs_v, idxd_v, rows_v, sem):
    wid = lax.axis_index("s") * NC + lax.axis_index("c")
    base = wid * EW
    pltpu.sync_copy(src_hbm.at[pl.ds(base, EWP)], idxs_v)
    pltpu.sync_copy(dst_hbm.at[pl.ds(base, EWP)], idxd_v)
    for oc in range(NOC):
        descs = [
            pltpu.async_copy(
                a_hbm.at[idxs_v.at[pl.ds(oc * OC + j * GC, GC)]],
                rows_v.at[pl.ds(j * GC, GC)], sem)
            for j in range(OC // GC)
        ]
        for d in descs:
            d.wait()
        descs = [
            pltpu.async_copy(
                b_hbm.at[idxd_v.at[pl.ds(oc * OC + j * GC, GC)]],
                rows_v.at[pl.ds(j * GC, GC)], sem, add=True)
            for j in range(OC // GC)
        ]
        for d in descs:
            d.wait()
        pltpu.sync_copy(rows_v, g_hbm.at[pl.ds(base + oc * OC, OC), pl.ds(0, H)])


@functools.partial(
    pl.kernel, mesh=_sc_mesh, compiler_params=_sc_params,
    out_type=jax.ShapeDtypeStruct((R, E), jnp.float32),
    scratch_types=[
        pltpu.VMEM((EC,), jnp.int32),
        pltpu.VMEM((EC,), jnp.int32),
        pltpu.VMEM((EC,), jnp.float32),
        pltpu.VMEM((EC,), jnp.float32),
        pltpu.VMEM((NR, 128), jnp.float32),
        pltpu.VMEM((NR, 128), jnp.float32),
        pltpu.VMEM((NR,), jnp.int32),
        pltpu.VMEM_SHARED((NR, 128), jnp.float32),
    ],
)
def _frontier(src_hbm, dst_hbm, p_hbm, m0_hbm, rows_hbm, out_hbm,
              srcv, dstv, pv, pmv, mloc, updv, rowids, sm):
    cid = lax.axis_index("c")
    sid = lax.axis_index("s")
    tbase = sid * EC
    pltpu.sync_copy(src_hbm.at[pl.ds(tbase, EC)], srcv)
    pltpu.sync_copy(dst_hbm.at[pl.ds(tbase, EC)], dstv)
    pltpu.sync_copy(p_hbm.at[pl.ds(tbase, EC)], pv)
    pltpu.sync_copy(m0_hbm, mloc)
    pltpu.sync_copy(rows_hbm, rowids)

    @pl.when(sid == 0)
    def _():
        pltpu.sync_copy(m0_hbm, sm)

    plsc.subcore_barrier()

    zeros = jnp.zeros((16,), jnp.float32)
    ones = jnp.ones((16,), jnp.float32)

    for r in range(R):
        def zbody(i, _):
            updv[i >> 3, pl.ds((i & 7) * 16, 16)] = zeros
            return 0
        lax.fori_loop(0, NR * 8, zbody, 0)

        def ebody(k, _):
            off = k * 16
            sv = srcv[pl.ds(off, 16)]
            dv = dstv[pl.ds(off, 16)]
            ppv = pv[pl.ds(off, 16)]
            ms = plsc.load_gather(mloc, [sv >> 7, sv & 127])
            md = plsc.load_gather(mloc, [dv >> 7, dv & 127])
            pm = ppv * ms * (1.0 - md)
            pmv[pl.ds(off, 16)] = pm
            plsc.store_scatter(updv, [dv >> 7, dv & 127], ones, mask=pm > 0.5)
            return 0
        lax.fori_loop(0, EC // 16, ebody, 0)

        pltpu.sync_copy(pmv.at[pl.ds(cid * HALF, HALF)],
                        out_hbm.at[r, pl.ds(tbase + cid * HALF, HALF)])

        if r < R - 1:
            pltpu.sync_copy(updv, sm.at[rowids], add=True)
            plsc.subcore_barrier()
            pltpu.sync_copy(sm, mloc)
            plsc.subcore_barrier()

            def cbody(i, _):
                v = mloc[i >> 3, pl.ds((i & 7) * 16, 16)]
                mloc[i >> 3, pl.ds((i & 7) * 16, 16)] = jnp.where(v > 0.5, 1.0, 0.0)
                return 0
            lax.fori_loop(0, NR * 8, cbody, 0)


def kernel(x, edge_index, edge_attr, W1, b1, W2, b2):
    src = edge_index[0]
    dst = edge_index[1]
    w1a = W1[:D]
    w1b = W1[D:2 * D]
    w1c = W1[2 * D:]

    a_tab, b_tab = pl.pallas_call(
        _ab_body,
        grid=(N // 1000,),
        in_specs=[
            pl.BlockSpec((1000, D), lambda i: (i, 0)),
            pl.BlockSpec((D, H), lambda i: (0, 0)),
            pl.BlockSpec((D, H), lambda i: (0, 0)),
        ],
        out_specs=[
            pl.BlockSpec((1000, H), lambda i: (i, 0)),
            pl.BlockSpec((1000, H), lambda i: (i, 0)),
        ],
        out_shape=[
            jax.ShapeDtypeStruct((N, H), jnp.float32),
            jax.ShapeDtypeStruct((N, H), jnp.float32),
        ],
    )(x, w1a, w1b)

    pad = jnp.zeros((EP - E,), jnp.int32)
    srcp = jnp.concatenate([src, pad])
    dstp = jnp.concatenate([dst, pad])
    g = _gather_ab(a_tab, b_tab, srcp, dstp)

    p2 = pl.pallas_call(
        _score_body,
        grid=(E // EB,),
        in_specs=[
            pl.BlockSpec((EB, 2 * H), lambda i: (i, 0)),
            pl.BlockSpec((EB, DE), lambda i: (i, 0)),
            pl.BlockSpec((DE, H), lambda i: (0, 0)),
            pl.BlockSpec((1, H), lambda i: (0, 0)),
            pl.BlockSpec((H, 1), lambda i: (0, 0)),
            pl.BlockSpec((1, 1), lambda i: (0, 0)),
        ],
        out_specs=pl.BlockSpec((EB, 1), lambda i: (i, 0)),
        out_shape=jax.ShapeDtypeStruct((E, 1), jnp.float32),
    )(g, edge_attr, w1c, b1.reshape(1, H), W2, b2.reshape(1, 1))
    p = p2.reshape(E)

    m0 = jnp.where(jnp.arange(NPAD) % 10 == 0, 1.0, 0.0)
    m0 = m0.astype(jnp.float32).reshape(NR, 128)
    rowids = jnp.arange(NR, dtype=jnp.int32)
    return _frontier(src, dst, p, m0, rowids)


# speedup vs baseline: 45.3920x; 1.0085x over previous
import functools

import jax
import jax.numpy as jnp
from jax import lax
from jax.experimental import pallas as pl
from jax.experimental.pallas import tpu as pltpu
from jax.experimental.pallas import tpu_sc as plsc

N = 10000
E = 320000
D = 128
DE = 16
H = 64
R = 5

NC, NS = 2, 16
NW = NC * NS
EH = E // 2
EW = EH // NW
GC = 128
OC = 1280
NOC = 4
EWP = NOC * OC
EHP = (NW - 1) * EW + EWP
EP = E + 256

EC = E // NS
HALF = EC // NC
NR = (N + 127) // 128
NPAD = NR * 128

EB = 4000


def _ab_body(x_ref, wa_ref, wb_ref, a_ref, b_ref):
    xb = x_ref[...]
    a_ref[...] = lax.dot(xb, wa_ref[...], preferred_element_type=jnp.float32)
    b_ref[...] = lax.dot(xb, wb_ref[...], preferred_element_type=jnp.float32)


def _score_body(g_ref, ea_ref, wc_ref, b1_ref, w2_ref, b2_ref, p_ref):
    h = g_ref[:, :H] + lax.dot(ea_ref[...], wc_ref[...],
                               preferred_element_type=jnp.float32) + b1_ref[...]
    h = jnp.maximum(h, 0.0)
    s = lax.dot(h, w2_ref[...], preferred_element_type=jnp.float32) + b2_ref[...]
    p_ref[...] = jax.nn.sigmoid(s)


_sc_mesh = plsc.VectorSubcoreMesh(core_axis_name="c", subcore_axis_name="s")
_sc_params = pltpu.CompilerParams(use_tc_tiling_on_sc=False,
                                  needs_layout_passes=False)


def _make_gather(edge_base):
    @functools.partial(
        pl.kernel, mesh=_sc_mesh, compiler_params=_sc_params,
        out_type=jax.ShapeDtypeStruct((EHP, 2 * H), jnp.float32),
        scratch_types=[
            pltpu.VMEM((EWP,), jnp.int32),
            pltpu.VMEM((EWP,), jnp.int32),
            pltpu.VMEM((OC, H), jnp.float32),
            pltpu.SemaphoreType.DMA,
        ],
    )
    def _gather_ab(a_hbm, b_hbm, src_hbm, dst_hbm, g_hbm,
                   idxs_v, idxd_v, rows_v, sem):
        wid = lax.axis_index("s") * NC + lax.axis_index("c")
        obase = wid * EW
        base = edge_base + obase
        pltpu.sync_copy(src_hbm.at[pl.ds(base, EWP)], idxs_v)
        pltpu.sync_copy(dst_hbm.at[pl.ds(base, EWP)], idxd_v)
        for oc in range(NOC):
            descs = [
                pltpu.async_copy(
                    a_hbm.at[idxs_v.at[pl.ds(oc * OC + j * GC, GC)]],
                    rows_v.at[pl.ds(j * GC, GC)], sem)
                for j in range(OC // GC)
            ]
            for d in descs:
                d.wait()
            descs = [
                pltpu.async_copy(
                    b_hbm.at[idxd_v.at[pl.ds(oc * OC + j * GC, GC)]],
                    rows_v.at[pl.ds(j * GC, GC)], sem, add=True)
                for j in range(OC // GC)
            ]
            for d in descs:
                d.wait()
            pltpu.sync_copy(rows_v,
                            g_hbm.at[pl.ds(obase + oc * OC, OC), pl.ds(0, H)])
    return _gather_ab


_gather_h1 = _make_gather(0)
_gather_h2 = _make_gather(EH)


@functools.partial(
    pl.kernel, mesh=_sc_mesh, compiler_params=_sc_params,
    out_type=jax.ShapeDtypeStruct((R, E), jnp.float32),
    scratch_types=[
        pltpu.VMEM((EC,), jnp.int32),
        pltpu.VMEM((EC,), jnp.int32),
        pltpu.VMEM((EC,), jnp.float32),
        pltpu.VMEM((EC,), jnp.float32),
        pltpu.VMEM((NR, 128), jnp.float32),
        pltpu.VMEM((NR, 128), jnp.float32),
        pltpu.VMEM((NR,), jnp.int32),
        pltpu.VMEM_SHARED((NR, 128), jnp.float32),
    ],
)
def _frontier(src_hbm, dst_hbm, p1_hbm, p2_hbm, m0_hbm, rows_hbm, out_hbm,
              srcv, dstv, pv, pmv, mloc, updv, rowids, sm):
    cid = lax.axis_index("c")
    sid = lax.axis_index("s")
    tbase = sid * EC
    pltpu.sync_copy(src_hbm.at[pl.ds(tbase, EC)], srcv)
    pltpu.sync_copy(dst_hbm.at[pl.ds(tbase, EC)], dstv)

    @pl.when(sid < NS // 2)
    def _():
        pltpu.sync_copy(p1_hbm.at[pl.ds(tbase, EC)], pv)

    @pl.when(sid >= NS // 2)
    def _():
        pltpu.sync_copy(p2_hbm.at[pl.ds(tbase - EH, EC)], pv)

    pltpu.sync_copy(m0_hbm, mloc)
    pltpu.sync_copy(rows_hbm, rowids)

    @pl.when(sid == 0)
    def _():
        pltpu.sync_copy(m0_hbm, sm)

    plsc.subcore_barrier()

    zeros = jnp.zeros((16,), jnp.float32)
    ones = jnp.ones((16,), jnp.float32)

    for r in range(R):
        def zbody(i, _):
            updv[i >> 3, pl.ds((i & 7) * 16, 16)] = zeros
            return 0
        lax.fori_loop(0, NR * 8, zbody, 0)

        def ebody(k, _):
            off = k * 16
            sv = srcv[pl.ds(off, 16)]
            dv = dstv[pl.ds(off, 16)]
            ppv = pv[pl.ds(off, 16)]
            ms = plsc.load_gather(mloc, [sv >> 7, sv & 127])
            md = plsc.load_gather(mloc, [dv >> 7, dv & 127])
            pm = ppv * ms * (1.0 - md)
            pmv[pl.ds(off, 16)] = pm
            plsc.store_scatter(updv, [dv >> 7, dv & 127], ones, mask=pm > 0.5)
            return 0
        lax.fori_loop(0, EC // 16, ebody, 0)

        pltpu.sync_copy(pmv.at[pl.ds(cid * HALF, HALF)],
                        out_hbm.at[r, pl.ds(tbase + cid * HALF, HALF)])

        if r < R - 1:
            pltpu.sync_copy(updv, sm.at[rowids], add=True)
            plsc.subcore_barrier()
            pltpu.sync_copy(sm, mloc)
            plsc.subcore_barrier()

            def cbody(i, _):
                v = mloc[i >> 3, pl.ds((i & 7) * 16, 16)]
                mloc[i >> 3, pl.ds((i & 7) * 16, 16)] = jnp.where(v > 0.5, 1.0, 0.0)
                return 0
            lax.fori_loop(0, NR * 8, cbody, 0)


def kernel(x, edge_index, edge_attr, W1, b1, W2, b2):
    src = edge_index[0]
    dst = edge_index[1]
    w1a = W1[:D]
    w1b = W1[D:2 * D]
    w1c = W1[2 * D:]

    a_tab, b_tab = pl.pallas_call(
        _ab_body,
        grid=(N // 1000,),
        in_specs=[
            pl.BlockSpec((1000, D), lambda i: (i, 0)),
            pl.BlockSpec((D, H), lambda i: (0, 0)),
            pl.BlockSpec((D, H), lambda i: (0, 0)),
        ],
        out_specs=[
            pl.BlockSpec((1000, H), lambda i: (i, 0)),
            pl.BlockSpec((1000, H), lambda i: (i, 0)),
        ],
        out_shape=[
            jax.ShapeDtypeStruct((N, H), jnp.float32),
            jax.ShapeDtypeStruct((N, H), jnp.float32),
        ],
    )(x, w1a, w1b)

    pad = jnp.zeros((EP - E,), jnp.int32)
    srcp = jnp.concatenate([src, pad])
    dstp = jnp.concatenate([dst, pad])
    g1 = _gather_h1(a_tab, b_tab, srcp, dstp)
    g2 = _gather_h2(a_tab, b_tab, srcp, dstp)

    def _score(g, ea_block_off):
        return pl.pallas_call(
            _score_body,
            grid=(EH // EB,),
            in_specs=[
                pl.BlockSpec((EB, 2 * H), lambda i: (i, 0)),
                pl.BlockSpec((EB, DE), lambda i: (i + ea_block_off, 0)),
                pl.BlockSpec((DE, H), lambda i: (0, 0)),
                pl.BlockSpec((1, H), lambda i: (0, 0)),
                pl.BlockSpec((H, 1), lambda i: (0, 0)),
                pl.BlockSpec((1, 1), lambda i: (0, 0)),
            ],
            out_specs=pl.BlockSpec((EB, 1), lambda i: (i, 0)),
            out_shape=jax.ShapeDtypeStruct((EH, 1), jnp.float32),
        )(g, edge_attr, w1c, b1.reshape(1, H), W2, b2.reshape(1, 1))

    p1 = _score(g1, 0).reshape(EH)
    p2 = _score(g2, EH // EB).reshape(EH)

    m0 = jnp.where(jnp.arange(NPAD) % 10 == 0, 1.0, 0.0)
    m0 = m0.astype(jnp.float32).reshape(NR, 128)
    rowids = jnp.arange(NR, dtype=jnp.int32)
    return _frontier(src, dst, p1, p2, m0, rowids)


# speedup vs baseline: 54.2528x; 1.1952x over previous
import functools

import jax
import jax.numpy as jnp
from jax import lax
from jax.experimental import pallas as pl
from jax.experimental.pallas import tpu as pltpu
from jax.experimental.pallas import tpu_sc as plsc

N = 10000
E = 320000
D = 128
DE = 16
H = 64
R = 5

NC, NS = 2, 16
NW = NC * NS
EH = E // 2
EW = EH // NW
GC = 128
OC = 1280
NOC = 4
EWP = NOC * OC
EHP = (NW - 1) * EW + EWP
EP = E + 256

EC = E // NS
HALF = EC // NC
NR = (N + 127) // 128
NPAD = NR * 128

EB = 6400


def _ab_body(x_ref, wa_ref, wb_ref, a_ref, b_ref):
    xb = x_ref[...]
    a_ref[...] = lax.dot(xb, wa_ref[...], preferred_element_type=jnp.float32)
    b_ref[...] = lax.dot(xb, wb_ref[...], preferred_element_type=jnp.float32)


def _score_body(g_ref, ea_ref, wc_ref, b1_ref, w2_ref, b2_ref, p_ref):
    c = lax.dot_general(ea_ref[...], wc_ref[...], (((0,), (0,)), ((), ())),
                        preferred_element_type=jnp.float32)
    h = g_ref[:, :H] + c + b1_ref[...]
    h = jnp.maximum(h, 0.0)
    s = lax.dot(h, w2_ref[...], preferred_element_type=jnp.float32) + b2_ref[...]
    p_ref[...] = jax.nn.sigmoid(s)


_sc_mesh = plsc.VectorSubcoreMesh(core_axis_name="c", subcore_axis_name="s")
_sc_params = pltpu.CompilerParams(use_tc_tiling_on_sc=False,
                                  needs_layout_passes=False)


def _make_gather(edge_base):
    @functools.partial(
        pl.kernel, mesh=_sc_mesh, compiler_params=_sc_params,
        out_type=jax.ShapeDtypeStruct((EHP, 2 * H), jnp.float32),
        scratch_types=[
            pltpu.VMEM((EWP,), jnp.int32),
            pltpu.VMEM((EWP,), jnp.int32),
            pltpu.VMEM((OC, H), jnp.float32),
            pltpu.SemaphoreType.DMA,
        ],
    )
    def _gather_ab(a_hbm, b_hbm, src_hbm, dst_hbm, g_hbm,
                   idxs_v, idxd_v, rows_v, sem):
        wid = lax.axis_index("s") * NC + lax.axis_index("c")
        obase = wid * EW
        base = edge_base + obase
        pltpu.sync_copy(src_hbm.at[pl.ds(base, EWP)], idxs_v)
        pltpu.sync_copy(dst_hbm.at[pl.ds(base, EWP)], idxd_v)
        for oc in range(NOC):
            descs = [
                pltpu.async_copy(
                    a_hbm.at[idxs_v.at[pl.ds(oc * OC + j * GC, GC)]],
                    rows_v.at[pl.ds(j * GC, GC)], sem)
                for j in range(OC // GC)
            ]
            for d in descs:
                d.wait()
            descs = [
                pltpu.async_copy(
                    b_hbm.at[idxd_v.at[pl.ds(oc * OC + j * GC, GC)]],
                    rows_v.at[pl.ds(j * GC, GC)], sem, add=True)
                for j in range(OC // GC)
            ]
            for d in descs:
                d.wait()
            pltpu.sync_copy(rows_v,
                            g_hbm.at[pl.ds(obase + oc * OC, OC), pl.ds(0, H)])
    return _gather_ab


_gather_h1 = _make_gather(0)
_gather_h2 = _make_gather(EH)


@functools.partial(
    pl.kernel, mesh=_sc_mesh, compiler_params=_sc_params,
    out_type=jax.ShapeDtypeStruct((R, E), jnp.float32),
    scratch_types=[
        pltpu.VMEM((EC,), jnp.int32),
        pltpu.VMEM((EC,), jnp.int32),
        pltpu.VMEM((EC,), jnp.float32),
        pltpu.VMEM((EC,), jnp.float32),
        pltpu.VMEM((NR, 128), jnp.float32),
        pltpu.VMEM((NR, 128), jnp.float32),
        pltpu.VMEM((NR,), jnp.int32),
        pltpu.VMEM_SHARED((NR, 128), jnp.float32),
    ],
)
def _frontier(src_hbm, dst_hbm, p1_hbm, p2_hbm, m0_hbm, rows_hbm, out_hbm,
              srcv, dstv, pv, pmv, mloc, updv, rowids, sm):
    cid = lax.axis_index("c")
    sid = lax.axis_index("s")
    tbase = sid * EC
    pltpu.sync_copy(src_hbm.at[pl.ds(tbase, EC)], srcv)
    pltpu.sync_copy(dst_hbm.at[pl.ds(tbase, EC)], dstv)

    @pl.when(sid < NS // 2)
    def _():
        pltpu.sync_copy(p1_hbm.at[pl.ds(tbase, EC)], pv)

    @pl.when(sid >= NS // 2)
    def _():
        pltpu.sync_copy(p2_hbm.at[pl.ds(tbase - EH, EC)], pv)

    pltpu.sync_copy(m0_hbm, mloc)
    pltpu.sync_copy(rows_hbm, rowids)

    @pl.when(sid == 0)
    def _():
        pltpu.sync_copy(m0_hbm, sm)

    plsc.subcore_barrier()

    zeros = jnp.zeros((16,), jnp.float32)
    ones = jnp.ones((16,), jnp.float32)

    for r in range(R):
        def zbody(i, _):
            updv[i >> 3, pl.ds((i & 7) * 16, 16)] = zeros
            return 0
        lax.fori_loop(0, NR * 8, zbody, 0)

        def ebody(k, _):
            off = k * 16
            sv = srcv[pl.ds(off, 16)]
            dv = dstv[pl.ds(off, 16)]
            ppv = pv[pl.ds(off, 16)]
            ms = plsc.load_gather(mloc, [sv >> 7, sv & 127])
            md = plsc.load_gather(mloc, [dv >> 7, dv & 127])
            pm = ppv * ms * (1.0 - md)
            pmv[pl.ds(off, 16)] = pm
            plsc.store_scatter(updv, [dv >> 7, dv & 127], ones, mask=pm > 0.5)
            return 0
        lax.fori_loop(0, EC // 16, ebody, 0)

        pltpu.sync_copy(pmv.at[pl.ds(cid * HALF, HALF)],
                        out_hbm.at[r, pl.ds(tbase + cid * HALF, HALF)])

        if r < R - 1:
            pltpu.sync_copy(updv, sm.at[rowids], add=True)
            plsc.subcore_barrier()
            pltpu.sync_copy(sm, mloc)
            plsc.subcore_barrier()

            def cbody(i, _):
                v = mloc[i >> 3, pl.ds((i & 7) * 16, 16)]
                mloc[i >> 3, pl.ds((i & 7) * 16, 16)] = jnp.where(v > 0.5, 1.0, 0.0)
                return 0
            lax.fori_loop(0, NR * 8, cbody, 0)


def kernel(x, edge_index, edge_attr, W1, b1, W2, b2):
    src = edge_index[0]
    dst = edge_index[1]
    w1a = W1[:D]
    w1b = W1[D:2 * D]
    w1c = W1[2 * D:]

    a_tab, b_tab = pl.pallas_call(
        _ab_body,
        grid=(N // 1000,),
        in_specs=[
            pl.BlockSpec((1000, D), lambda i: (i, 0)),
            pl.BlockSpec((D, H), lambda i: (0, 0)),
            pl.BlockSpec((D, H), lambda i: (0, 0)),
        ],
        out_specs=[
            pl.BlockSpec((1000, H), lambda i: (i, 0)),
            pl.BlockSpec((1000, H), lambda i: (i, 0)),
        ],
        out_shape=[
            jax.ShapeDtypeStruct((N, H), jnp.float32),
            jax.ShapeDtypeStruct((N, H), jnp.float32),
        ],
    )(x, w1a, w1b)

    pad = jnp.zeros((EP - E,), jnp.int32)
    srcp = jnp.concatenate([src, pad])
    dstp = jnp.concatenate([dst, pad])
    g1 = _gather_h1(a_tab, b_tab, srcp, dstp)
    g2 = _gather_h2(a_tab, b_tab, srcp, dstp)

    def _score(g, ea_block_off):
        return pl.pallas_call(
            _score_body,
            grid=(EH // EB,),
            in_specs=[
                pl.BlockSpec((EB, 2 * H), lambda i: (i, 0)),
                pl.BlockSpec((DE, EB), lambda i: (0, i + ea_block_off)),
                pl.BlockSpec((DE, H), lambda i: (0, 0)),
                pl.BlockSpec((1, H), lambda i: (0, 0)),
                pl.BlockSpec((H, 1), lambda i: (0, 0)),
                pl.BlockSpec((1, 1), lambda i: (0, 0)),
            ],
            out_specs=pl.BlockSpec((EB, 1), lambda i: (i, 0)),
            out_shape=jax.ShapeDtypeStruct((EH, 1), jnp.float32),
        )(g, edge_attr.T, w1c, b1.reshape(1, H), W2, b2.reshape(1, 1))

    p1 = _score(g1, 0).reshape(EH)
    p2 = _score(g2, EH // EB).reshape(EH)

    m0 = jnp.where(jnp.arange(NPAD) % 10 == 0, 1.0, 0.0)
    m0 = m0.astype(jnp.float32).reshape(NR, 128)
    rowids = jnp.arange(NR, dtype=jnp.int32)
    return _frontier(src, dst, p1, p2, m0, rowids)


# speedup vs baseline: 58.2490x; 1.0737x over previous
import functools

import jax
import jax.numpy as jnp
from jax import lax
from jax.experimental import pallas as pl
from jax.experimental.pallas import tpu as pltpu
from jax.experimental.pallas import tpu_sc as plsc

N = 10000
E = 320000
D = 128
DE = 16
H = 64
R = 5

NC, NS = 2, 16
NW = NC * NS
EH = E // 2
EW = EH // NW
GC = 128
OC = 1280
NOC = 4
EWP = NOC * OC
EHP = (NW - 1) * EW + EWP
EP = E + 256

EC = E // NS
HALF = EC // NC
NR = (N + 127) // 128
NPAD = NR * 128

EB = 6400


def _ab_body(x_ref, wa_ref, wb_ref, a_ref, b_ref):
    xb = x_ref[...]
    a_ref[...] = lax.dot(xb, wa_ref[...], preferred_element_type=jnp.float32)
    b_ref[...] = lax.dot(xb, wb_ref[...], preferred_element_type=jnp.float32)


def _score_body(g_ref, ea_ref, wc_ref, b1_ref, w2_ref, b2_ref, p_ref):
    c = lax.dot_general(ea_ref[...], wc_ref[...], (((0,), (0,)), ((), ())),
                        preferred_element_type=jnp.float32)
    h = g_ref[:, :H] + c + b1_ref[...]
    h = jnp.maximum(h, 0.0)
    s = lax.dot(h, w2_ref[...], preferred_element_type=jnp.float32) + b2_ref[...]
    p_ref[...] = jax.nn.sigmoid(s)


_sc_mesh = plsc.VectorSubcoreMesh(core_axis_name="c", subcore_axis_name="s")
_sc_params = pltpu.CompilerParams(use_tc_tiling_on_sc=False,
                                  needs_layout_passes=False)


def _make_gather(edge_base):
    @functools.partial(
        pl.kernel, mesh=_sc_mesh, compiler_params=_sc_params,
        out_type=jax.ShapeDtypeStruct((EHP, 2 * H), jnp.float32),
        scratch_types=[
            pltpu.VMEM((EWP,), jnp.int32),
            pltpu.VMEM((EWP,), jnp.int32),
            pltpu.VMEM((OC, H), jnp.float32),
            pltpu.SemaphoreType.DMA,
        ],
    )
    def _gather_ab(a_hbm, b_hbm, src_hbm, dst_hbm, g_hbm,
                   idxs_v, idxd_v, rows_v, sem):
        wid = lax.axis_index("s") * NC + lax.axis_index("c")
        obase = wid * EW
        base = edge_base + obase
        pltpu.sync_copy(src_hbm.at[pl.ds(base, EWP)], idxs_v)
        pltpu.sync_copy(dst_hbm.at[pl.ds(base, EWP)], idxd_v)
        for oc in range(NOC):
            descs = [
                pltpu.async_copy(
                    a_hbm.at[idxs_v.at[pl.ds(oc * OC + j * GC, GC)]],
                    rows_v.at[pl.ds(j * GC, GC)], sem)
                for j in range(OC // GC)
            ]
            for d in descs:
                d.wait()
            descs = [
                pltpu.async_copy(
                    b_hbm.at[idxd_v.at[pl.ds(oc * OC + j * GC, GC)]],
                    rows_v.at[pl.ds(j * GC, GC)], sem, add=True)
                for j in range(OC // GC)
            ]
            for d in descs:
                d.wait()
            pltpu.sync_copy(rows_v,
                            g_hbm.at[pl.ds(obase + oc * OC, OC), pl.ds(0, H)])
    return _gather_ab


_gather_h1 = _make_gather(0)
_gather_h2 = _make_gather(EH)


@functools.partial(
    pl.kernel, mesh=_sc_mesh, compiler_params=_sc_params,
    out_type=jax.ShapeDtypeStruct((R, E), jnp.float32),
    scratch_types=[
        pltpu.VMEM((EC,), jnp.int32),
        pltpu.VMEM((EC,), jnp.int32),
        pltpu.VMEM((EC,), jnp.float32),
        pltpu.VMEM((EC,), jnp.float32),
        pltpu.VMEM((NR, 128), jnp.float32),
        pltpu.VMEM((NR, 128), jnp.float32),
        pltpu.VMEM((NR,), jnp.int32),
        pltpu.VMEM_SHARED((NR, 128), jnp.float32),
    ],
)
def _frontier(src_hbm, dst_hbm, p1_hbm, p2_hbm, m0_hbm, rows_hbm, out_hbm,
              srcv, dstv, pv, pmv, mloc, updv, rowids, sm):
    cid = lax.axis_index("c")
    sid = lax.axis_index("s")
    tbase = sid * EC
    pltpu.sync_copy(src_hbm.at[pl.ds(tbase, EC)], srcv)
    pltpu.sync_copy(dst_hbm.at[pl.ds(tbase, EC)], dstv)

    @pl.when(sid < NS // 2)
    def _():
        pltpu.sync_copy(p1_hbm.at[pl.ds(tbase, EC)], pv)

    @pl.when(sid >= NS // 2)
    def _():
        pltpu.sync_copy(p2_hbm.at[pl.ds(tbase - EH, EC)], pv)

    pltpu.sync_copy(m0_hbm, mloc)
    pltpu.sync_copy(rows_hbm, rowids)

    @pl.when(sid == 0)
    def _():
        pltpu.sync_copy(m0_hbm, sm)

    plsc.subcore_barrier()

    zeros = jnp.zeros((16,), jnp.float32)
    ones = jnp.ones((16,), jnp.float32)

    def zbody(i, _):
        updv[i >> 3, pl.ds((i & 7) * 16, 16)] = zeros
        return 0
    lax.fori_loop(0, NR * 8, zbody, 0)

    for r in range(R):
        def ebody(k, _):
            for u in range(5):
                off = k * 80 + u * 16
                sv = srcv[pl.ds(off, 16)]
                dv = dstv[pl.ds(off, 16)]
                ppv = pv[pl.ds(off, 16)]
                ms = plsc.load_gather(mloc, [sv >> 7, sv & 127])
                md = plsc.load_gather(mloc, [dv >> 7, dv & 127])
                live = (ms > 0.5) & (md < 0.5)
                pm = jnp.where(live, ppv, 0.0)
                pmv[pl.ds(off, 16)] = pm
                plsc.store_scatter(updv, [dv >> 7, dv & 127], ones,
                                   mask=pm > 0.5)
            return 0
        lax.fori_loop(0, EC // 80, ebody, 0)

        pltpu.sync_copy(pmv.at[pl.ds(cid * HALF, HALF)],
                        out_hbm.at[r, pl.ds(tbase + cid * HALF, HALF)])

        if r < R - 1:
            pltpu.sync_copy(updv, sm.at[rowids], add=True)
            plsc.subcore_barrier()
            pltpu.sync_copy(sm, mloc)
            plsc.subcore_barrier()


def kernel(x, edge_index, edge_attr, W1, b1, W2, b2):
    src = edge_index[0]
    dst = edge_index[1]
    w1a = W1[:D]
    w1b = W1[D:2 * D]
    w1c = W1[2 * D:]

    a_tab, b_tab = pl.pallas_call(
        _ab_body,
        grid=(N // 1000,),
        in_specs=[
            pl.BlockSpec((1000, D), lambda i: (i, 0)),
            pl.BlockSpec((D, H), lambda i: (0, 0)),
            pl.BlockSpec((D, H), lambda i: (0, 0)),
        ],
        out_specs=[
            pl.BlockSpec((1000, H), lambda i: (i, 0)),
            pl.BlockSpec((1000, H), lambda i: (i, 0)),
        ],
        out_shape=[
            jax.ShapeDtypeStruct((N, H), jnp.float32),
            jax.ShapeDtypeStruct((N, H), jnp.float32),
        ],
    )(x, w1a, w1b)

    pad = jnp.zeros((EP - E,), jnp.int32)
    srcp = jnp.concatenate([src, pad])
    dstp = jnp.concatenate([dst, pad])
    g1 = _gather_h1(a_tab, b_tab, srcp, dstp)
    g2 = _gather_h2(a_tab, b_tab, srcp, dstp)

    def _score(g, ea_block_off):
        return pl.pallas_call(
            _score_body,
            grid=(EH // EB,),
            in_specs=[
                pl.BlockSpec((EB, 2 * H), lambda i: (i, 0)),
                pl.BlockSpec((DE, EB), lambda i: (0, i + ea_block_off)),
                pl.BlockSpec((DE, H), lambda i: (0, 0)),
                pl.BlockSpec((1, H), lambda i: (0, 0)),
                pl.BlockSpec((H, 1), lambda i: (0, 0)),
                pl.BlockSpec((1, 1), lambda i: (0, 0)),
            ],
            out_specs=pl.BlockSpec((EB, 1), lambda i: (i, 0)),
            out_shape=jax.ShapeDtypeStruct((EH, 1), jnp.float32),
        )(g, edge_attr.T, w1c, b1.reshape(1, H), W2, b2.reshape(1, 1))

    p1 = _score(g1, 0).reshape(EH)
    p2 = _score(g2, EH // EB).reshape(EH)

    m0 = jnp.where(jnp.arange(NPAD) % 10 == 0, 1.0, 0.0)
    m0 = m0.astype(jnp.float32).reshape(NR, 128)
    rowids = jnp.arange(NR, dtype=jnp.int32)
    return _frontier(src, dst, p1, p2, m0, rowids)


# speedup vs baseline: 68.3739x; 1.1738x over previous
import functools

import jax
import jax.numpy as jnp
from jax import lax
from jax.experimental import pallas as pl
from jax.experimental.pallas import tpu as pltpu
from jax.experimental.pallas import tpu_sc as plsc

N = 10000
E = 320000
D = 128
DE = 16
H = 64
R = 5

NC, NS = 2, 16
NW = NC * NS
EH = E // 2
EW = EH // NW
GC = 128
OC = 1280
NOC = 4
EWP = NOC * OC
EHP = (NW - 1) * EW + EWP
EP = E + 256

EC = E // NS
HALF = EC // NC
NR = (N + 127) // 128
NPAD = NR * 128

EB = 6400


def _ab_body(x_ref, wa_ref, wb_ref, a_ref, b_ref):
    xb = x_ref[...]
    a_ref[...] = lax.dot(xb, wa_ref[...], preferred_element_type=jnp.float32)
    b_ref[...] = lax.dot(xb, wb_ref[...], preferred_element_type=jnp.float32)


def _score_body(g_ref, ea_ref, wc_ref, b1_ref, w2_ref, b2_ref, p_ref):
    c = lax.dot_general(ea_ref[...], wc_ref[...], (((0,), (0,)), ((), ())),
                        preferred_element_type=jnp.float32)
    h = g_ref[:, :H] + c + b1_ref[...]
    h = jnp.maximum(h, 0.0)
    s = lax.dot(h, w2_ref[...], preferred_element_type=jnp.float32) + b2_ref[...]
    p_ref[...] = jax.nn.sigmoid(s)


_sc_mesh = plsc.VectorSubcoreMesh(core_axis_name="c", subcore_axis_name="s")
_sc_params = pltpu.CompilerParams(use_tc_tiling_on_sc=False,
                                  needs_layout_passes=False)


def _make_gather(edge_base):
    @functools.partial(
        pl.kernel, mesh=_sc_mesh, compiler_params=_sc_params,
        out_type=jax.ShapeDtypeStruct((EHP, 2 * H), jnp.float32),
        scratch_types=[
            pltpu.VMEM((EWP,), jnp.int32),
            pltpu.VMEM((EWP,), jnp.int32),
            pltpu.VMEM((OC, H), jnp.float32),
            pltpu.SemaphoreType.DMA,
        ],
    )
    def _gather_ab(a_hbm, b_hbm, src_hbm, dst_hbm, g_hbm,
                   idxs_v, idxd_v, rows_v, sem):
        wid = lax.axis_index("s") * NC + lax.axis_index("c")
        obase = wid * EW
        base = edge_base + obase
        pltpu.sync_copy(src_hbm.at[pl.ds(base, EWP)], idxs_v)
        pltpu.sync_copy(dst_hbm.at[pl.ds(base, EWP)], idxd_v)
        for oc in range(NOC):
            descs = [
                pltpu.async_copy(
                    a_hbm.at[idxs_v.at[pl.ds(oc * OC + j * GC, GC)]],
                    rows_v.at[pl.ds(j * GC, GC)], sem)
                for j in range(OC // GC)
            ]
            for d in descs:
                d.wait()
            descs = [
                pltpu.async_copy(
                    b_hbm.at[idxd_v.at[pl.ds(oc * OC + j * GC, GC)]],
                    rows_v.at[pl.ds(j * GC, GC)], sem, add=True)
                for j in range(OC // GC)
            ]
            for d in descs:
                d.wait()
            pltpu.sync_copy(rows_v,
                            g_hbm.at[pl.ds(obase + oc * OC, OC), pl.ds(0, H)])
    return _gather_ab


_gather_h1 = _make_gather(0)
_gather_h2 = _make_gather(EH)


@functools.partial(
    pl.kernel, mesh=_sc_mesh, compiler_params=_sc_params,
    out_type=jax.ShapeDtypeStruct((R, E), jnp.float32),
    scratch_types=[
        pltpu.VMEM((EC,), jnp.int32),
        pltpu.VMEM((EC,), jnp.int32),
        pltpu.VMEM((EC,), jnp.float32),
        pltpu.VMEM((EC,), jnp.float32),
        pltpu.VMEM((NR, 128), jnp.float32),
        pltpu.VMEM((NR, 128), jnp.float32),
        pltpu.VMEM((NR,), jnp.int32),
        pltpu.VMEM_SHARED((NR, 128), jnp.float32),
    ],
)
def _frontier(src_hbm, dst_hbm, p1_hbm, p2_hbm, m0_hbm, rows_hbm, out_hbm,
              srcv, dstv, pv, pmv, mloc, updv, rowids, sm):
    cid = lax.axis_index("c")
    sid = lax.axis_index("s")
    tbase = sid * EC
    pltpu.sync_copy(src_hbm.at[pl.ds(tbase, EC)], srcv)
    pltpu.sync_copy(dst_hbm.at[pl.ds(tbase, EC)], dstv)

    @pl.when(sid < NS // 2)
    def _():
        pltpu.sync_copy(p1_hbm.at[pl.ds(tbase, EC)], pv)

    @pl.when(sid >= NS // 2)
    def _():
        pltpu.sync_copy(p2_hbm.at[pl.ds(tbase - EH, EC)], pv)

    pltpu.sync_copy(m0_hbm, mloc)
    pltpu.sync_copy(rows_hbm, rowids)

    @pl.when(sid == 0)
    def _():
        pltpu.sync_copy(m0_hbm, sm)

    plsc.subcore_barrier()

    zeros = jnp.zeros((16,), jnp.float32)
    ones = jnp.ones((16,), jnp.float32)

    def zbody(i, _):
        updv[i >> 3, pl.ds((i & 7) * 16, 16)] = zeros
        return 0
    lax.fori_loop(0, NR * 8, zbody, 0)

    U = 10
    for r in range(R):
        def ebody(k, _):
            offs = [k * (U * 16) + u * 16 for u in range(U)]
            svs = [srcv[pl.ds(o, 16)] for o in offs]
            dvs = [dstv[pl.ds(o, 16)] for o in offs]
            pps = [pv[pl.ds(o, 16)] for o in offs]
            mss = [plsc.load_gather(mloc, [sv >> 7, sv & 127]) for sv in svs]
            mds = [plsc.load_gather(mloc, [dv >> 7, dv & 127]) for dv in dvs]
            pms = [jnp.where((ms > 0.5) & (md < 0.5), pp, 0.0)
                   for ms, md, pp in zip(mss, mds, pps)]
            for u in range(U):
                pmv[pl.ds(offs[u], 16)] = pms[u]
            for u in range(U):
                plsc.store_scatter(updv, [dvs[u] >> 7, dvs[u] & 127], ones,
                                   mask=pms[u] > 0.5)
            return 0
        lax.fori_loop(0, EC // (U * 16), ebody, 0)

        pltpu.sync_copy(pmv.at[pl.ds(cid * HALF, HALF)],
                        out_hbm.at[r, pl.ds(tbase + cid * HALF, HALF)])

        if r < R - 1:
            pltpu.sync_copy(updv, sm.at[rowids], add=True)
            plsc.subcore_barrier()
            pltpu.sync_copy(sm, mloc)
            plsc.subcore_barrier()


def kernel(x, edge_index, edge_attr, W1, b1, W2, b2):
    src = edge_index[0]
    dst = edge_index[1]
    w1a = W1[:D]
    w1b = W1[D:2 * D]
    w1c = W1[2 * D:]

    a_tab, b_tab = pl.pallas_call(
        _ab_body,
        grid=(N // 1000,),
        in_specs=[
            pl.BlockSpec((1000, D), lambda i: (i, 0)),
            pl.BlockSpec((D, H), lambda i: (0, 0)),
            pl.BlockSpec((D, H), lambda i: (0, 0)),
        ],
        out_specs=[
            pl.BlockSpec((1000, H), lambda i: (i, 0)),
            pl.BlockSpec((1000, H), lambda i: (i, 0)),
        ],
        out_shape=[
            jax.ShapeDtypeStruct((N, H), jnp.float32),
            jax.ShapeDtypeStruct((N, H), jnp.float32),
        ],
    )(x, w1a, w1b)

    pad = jnp.zeros((EP - E,), jnp.int32)
    srcp = jnp.concatenate([src, pad])
    dstp = jnp.concatenate([dst, pad])
    g1 = _gather_h1(a_tab, b_tab, srcp, dstp)
    g2 = _gather_h2(a_tab, b_tab, srcp, dstp)

    def _score(g, ea_block_off):
        return pl.pallas_call(
            _score_body,
            grid=(EH // EB,),
            in_specs=[
                pl.BlockSpec((EB, 2 * H), lambda i: (i, 0)),
                pl.BlockSpec((DE, EB), lambda i: (0, i + ea_block_off)),
                pl.BlockSpec((DE, H), lambda i: (0, 0)),
                pl.BlockSpec((1, H), lambda i: (0, 0)),
                pl.BlockSpec((H, 1), lambda i: (0, 0)),
                pl.BlockSpec((1, 1), lambda i: (0, 0)),
            ],
            out_specs=pl.BlockSpec((EB, 1), lambda i: (i, 0)),
            out_shape=jax.ShapeDtypeStruct((EH, 1), jnp.float32),
        )(g, edge_attr.T, w1c, b1.reshape(1, H), W2, b2.reshape(1, 1))

    p1 = _score(g1, 0).reshape(EH)
    p2 = _score(g2, EH // EB).reshape(EH)

    m0 = jnp.where(jnp.arange(NPAD) % 10 == 0, 1.0, 0.0)
    m0 = m0.astype(jnp.float32).reshape(NR, 128)
    rowids = jnp.arange(NR, dtype=jnp.int32)
    return _frontier(src, dst, p1, p2, m0, rowids)
